# Initial kernel scaffold; baseline (speedup 1.0000x reference)
import jax
import jax.numpy as jnp
from jax.experimental import pallas as pl


def kernel(x, params):
    raise NotImplementedError("write your pallas kernel here")


# speedup vs baseline: 95.5730x; 95.5730x over previous
import functools

import jax
import jax.numpy as jnp
from jax.experimental import pallas as pl
from jax.experimental.pallas import tpu as pltpu

_B = 32
_NEG_INV = -1.0 / 20.0
_BT = {3072: 2, 768: 4, 192: 8}
_OFFS = tuple(range(1, 11)) + tuple(-k for k in range(1, 11))


def _lap_valid(y):
    out_len = y.shape[1] - 20
    acc = None
    for off in _OFFS:
        msg = y[:, 10 + off:10 + off + out_len] * _NEG_INV
        acc = msg if acc is None else acc + msg
    return acc


def _bn_relu(ext, mean, std, g, be):
    return jnp.maximum(((ext - mean[None]) / std[None]) * g[None] + be[None], 0.0)


def _conv_body(src_cfg, cout, n, bt, want_loc, *refs):
    it = iter(refs)
    parts = []
    loc_val = None
    for (c_eff, has_norm, is_pool) in src_cfg:
        ext = next(it)[...]
        if has_norm:
            mean = next(it)[...]
            std = next(it)[...]
            g = next(it)[...]
            be = next(it)[...]
            ext = _bn_relu(ext, mean, std, g, be)
        if is_pool:
            grp = [ext[:, :, j * c_eff:(j + 1) * c_eff] for j in range(4)]
            vals = jnp.maximum(jnp.maximum(grp[0], grp[1]),
                               jnp.maximum(grp[2], grp[3]))
            lv = jnp.full(vals.shape, 3, jnp.int32)
            lv = jnp.where(grp[2] == vals, 2, lv)
            lv = jnp.where(grp[1] == vals, 1, lv)
            lv = jnp.where(grp[0] == vals, 0, lv)
            loc_val = lv
            ext = vals
        parts.append(ext)
    x0 = parts[0] if len(parts) == 1 else jnp.concatenate(parts, axis=2)
    cin = x0.shape[2]

    w_ref = next(it)
    b_ref = next(it)
    out_ref = next(it)
    rest = list(it)
    loc_ref = rest[0] if want_loc else None
    s0_ref, s1_ref, s2_ref = rest[-3:]

    x0e = jnp.concatenate([x0[:, n - 20:], x0, x0[:, :20]], axis=1)
    x1e = _lap_valid(x0e)
    x1 = x1e[:, 10:10 + n]
    x2 = 2.0 * _lap_valid(x1e) - x0

    def mm(xx, w):
        return jax.lax.dot_general(
            xx.reshape(bt * n, cin).astype(jnp.bfloat16),
            w.astype(jnp.bfloat16),
            (((1,), (0,)), ((), ())), preferred_element_type=jnp.float32)

    s0_ref[...] = mm(x0, w_ref[0])
    s1_ref[...] = mm(x1, w_ref[1])
    s2_ref[...] = mm(x2, w_ref[2])
    o = ((s0_ref[...] + s1_ref[...]) + s2_ref[...]) + b_ref[0][None, :]
    out_ref[...] = o.reshape(bt, n, cout)

    if want_loc:
        loc_ref[...] = loc_val


def _conv_block(n, srcs, W, b, pool_channels=None):
    bt = _BT[n]
    nb = _B // bt
    cout = W.shape[2]
    src_cfg = []
    in_specs = []
    args = []
    for s in srcs:
        arr = s['arr']
        craw = arr.shape[2]
        in_specs.append(pl.BlockSpec((bt, n, craw), lambda i: (i, 0, 0)))
        args.append(arr)
        has_norm = s.get('bn') is not None
        if has_norm:
            for v in s['bn']:
                in_specs.append(pl.BlockSpec((1, craw), lambda i: (0, 0)))
                args.append(v)
        src_cfg.append((s['C'], has_norm, bool(s.get('pool'))))
    cin = sum(c for (c, _, _) in src_cfg)
    in_specs.append(pl.BlockSpec((3, cin, cout), lambda i: (0, 0, 0)))
    in_specs.append(pl.BlockSpec((1, cout), lambda i: (0, 0)))
    args.extend([W, b.reshape(1, cout)])

    out_shapes = [jax.ShapeDtypeStruct((_B, n, cout), jnp.float32)]
    out_specs = [pl.BlockSpec((bt, n, cout), lambda i: (i, 0, 0))]
    want_loc = pool_channels is not None
    if want_loc:
        out_shapes.append(jax.ShapeDtypeStruct((_B, n, pool_channels), jnp.int32))
        out_specs.append(pl.BlockSpec((bt, n, pool_channels), lambda i: (i, 0, 0)))

    body = functools.partial(_conv_body, tuple(src_cfg), cout, n, bt,
                             want_loc)
    outs = pl.pallas_call(
        body,
        grid=(nb,),
        in_specs=in_specs,
        out_specs=out_specs,
        out_shape=out_shapes,
        scratch_shapes=[pltpu.VMEM((bt * n, cout), jnp.float32)] * 3,
    )(*args)
    return outs


def _unpool_body(c_ch, raw_ref, mean_ref, std_ref, g_ref, be_ref, loc_ref,
                 out_ref):
    xn = _bn_relu(raw_ref[...], mean_ref[...], std_ref[...], g_ref[...],
                  be_ref[...])
    loc = loc_ref[...]
    for j in range(4):
        out_ref[:, :, j * c_ch:(j + 1) * c_ch] = jnp.where(loc == j, xn, 0.0)


def _unpool(raw, bn, loc):
    b, q, c = raw.shape
    bt = 8
    out = pl.pallas_call(
        functools.partial(_unpool_body, c),
        grid=(b // bt,),
        in_specs=[
            pl.BlockSpec((bt, q, c), lambda i: (i, 0, 0)),
            pl.BlockSpec((1, c), lambda i: (0, 0)),
            pl.BlockSpec((1, c), lambda i: (0, 0)),
            pl.BlockSpec((1, c), lambda i: (0, 0)),
            pl.BlockSpec((1, c), lambda i: (0, 0)),
            pl.BlockSpec((bt, q, c), lambda i: (i, 0, 0)),
        ],
        out_specs=pl.BlockSpec((bt, q, 4 * c), lambda i: (i, 0, 0)),
        out_shape=jax.ShapeDtypeStruct((b, q, 4 * c), jnp.float32),
    )(raw, bn[0], bn[1], bn[2], bn[3], loc)
    return out.reshape(b, 4 * q, c)


def _finalize(raw, g, be):
    mean = jnp.mean(raw, axis=(0, 1))
    var = jnp.var(raw, axis=(0, 1))
    std = jnp.sqrt(var + 1e-5)
    return (mean.reshape(1, -1), std.reshape(1, -1),
            g.reshape(1, -1), be.reshape(1, -1))


def _tile4(bn):
    return tuple(jnp.tile(v, (1, 4)) for v in bn)


def kernel(x, params):
    p = params

    def fin(raw, name):
        return _finalize(raw, p[name]['g'], p[name]['be'])

    (r11,) = _conv_block(3072, [dict(arr=x, C=16)],
                         p['c11']['W'], p['c11']['b'])
    bn = fin(r11, 'c11')
    (r12,) = _conv_block(3072, [dict(arr=r11, bn=bn, C=16)],
                         p['c12']['W'], p['c12']['b'])
    bn = fin(r12, 'c12')
    (r13,) = _conv_block(3072, [dict(arr=r12, bn=bn, C=32)],
                         p['c13']['W'], p['c13']['b'])
    bn13 = fin(r13, 'c13')

    v13 = r13.reshape(_B, 768, 256)
    r21, loc1 = _conv_block(
        768, [dict(arr=v13, bn=_tile4(bn13), pool=True, C=64)],
        p['c21']['W'], p['c21']['b'], pool_channels=64)
    bn = fin(r21, 'c21')
    (r22,) = _conv_block(768, [dict(arr=r21, bn=bn, C=88)],
                         p['c22']['W'], p['c22']['b'])
    bn = fin(r22, 'c22')
    (r23,) = _conv_block(768, [dict(arr=r22, bn=bn, C=110)],
                         p['c23']['W'], p['c23']['b'])
    bn23 = fin(r23, 'c23')

    v23 = r23.reshape(_B, 192, 512)
    r31, loc2 = _conv_block(
        192, [dict(arr=v23, bn=_tile4(bn23), pool=True, C=128)],
        p['c31']['W'], p['c31']['b'], pool_channels=128)
    bn = fin(r31, 'c31')
    (r32,) = _conv_block(192, [dict(arr=r31, bn=bn, C=256)],
                         p['c32']['W'], p['c32']['b'])
    bn = fin(r32, 'c32')
    (r33,) = _conv_block(192, [dict(arr=r32, bn=bn, C=256)],
                         p['c33']['W'], p['c33']['b'])
    bn33 = fin(r33, 'c33')

    up1 = _unpool(r33, bn33, loc2)
    (r_d21,) = _conv_block(
        768,
        [dict(arr=up1, C=128),
         dict(arr=r23, bn=bn23, C=128)],
        p['d21']['W'], p['d21']['b'])
    bn = fin(r_d21, 'd21')
    (r_d22,) = _conv_block(768, [dict(arr=r_d21, bn=bn, C=128)],
                           p['d22']['W'], p['d22']['b'])
    bn_d22 = fin(r_d22, 'd22')

    up2 = _unpool(r_d22, bn_d22, loc1)
    (r_d11,) = _conv_block(
        3072,
        [dict(arr=up2, C=64),
         dict(arr=r13, bn=bn13, C=64)],
        p['d11']['W'], p['d11']['b'])
    bn = fin(r_d11, 'd11')
    (r_d12,) = _conv_block(3072, [dict(arr=r_d11, bn=bn, C=64)],
                           p['d12']['W'], p['d12']['b'])
    bn = fin(r_d12, 'd12')

    (y,) = _conv_block(3072, [dict(arr=r_d12, bn=bn, C=32)],
                       p['d13']['W'], p['d13']['b'])
    return y
